# Initial kernel scaffold; baseline (speedup 1.0000x reference)
import jax
import jax.numpy as jnp
from jax import lax
from jax.experimental import pallas as pl
from jax.experimental.pallas import tpu as pltpu
from jax.experimental.pallas import tpu_sc as plsc

NUM_EMB = 1000000
EMB_DIM = 32
BATCH = 16384
FIELDS = 26

NC = 2
NS = 16
NW = NC * NS
LANES = 16

B_FLAT = BATCH * FIELDS
B_PER_W = B_FLAT // NW
CHUNK = 1024
N_CHUNKS = B_PER_W // CHUNK
GROUPS = CHUNK // LANES

EPS = 1e-4
INV_SQRT_DIM = float(1.0 / (EMB_DIM ** 0.5))


def _normalize_chunk(buf):
    def group_body(g, carry):
        rid = g * LANES + lax.iota(jnp.int32, LANES)
        cols = []
        sq = jnp.zeros((LANES,), jnp.float32)
        for c in range(EMB_DIM):
            cid = jnp.full((LANES,), c, jnp.int32)
            xc = plsc.load_gather(buf, [rid, cid])
            cols.append(xc)
            sq = sq + xc * xc
        sq_m = jnp.maximum(sq, 1e-30)
        i = plsc.bitcast(sq_m, jnp.int32)
        i = jnp.int32(0x5F3759DF) - lax.shift_right_logical(i, 1)
        y = plsc.bitcast(i, jnp.float32)
        h = 0.5 * sq_m
        for _ in range(3):
            y = y * (1.5 - h * y * y)
        sqrt_s = sq_m * y
        scale = 1.0 / (EPS + sqrt_s * INV_SQRT_DIM)
        for c in range(EMB_DIM):
            cid = jnp.full((LANES,), c, jnp.int32)
            plsc.store_scatter(buf, [rid, cid], cols[c] * scale)
        return carry

    lax.fori_loop(0, GROUPS, group_body, 0)


def _sc_body(idx_hbm, table_hbm, out_hbm, idx_v, buf0, buf1, gsem0, gsem1,
             wsem0, wsem1):
    wid = lax.axis_index("s") * NC + lax.axis_index("c")
    base = wid * B_PER_W
    pltpu.sync_copy(idx_hbm.at[pl.ds(base, B_PER_W)], idx_v)

    bufs = (buf0, buf1)
    gsems = (gsem0, gsem1)
    wsems = (wsem0, wsem1)

    def gather(ci):
        return pltpu.async_copy(
            table_hbm.at[idx_v.at[pl.ds(ci * CHUNK, CHUNK)]],
            bufs[ci % 2], gsems[ci % 2])

    def writeback(ci):
        return pltpu.async_copy(
            bufs[ci % 2], out_hbm.at[pl.ds(base + ci * CHUNK, CHUNK)],
            wsems[ci % 2])

    g_copies = {0: gather(0), 1: gather(1)}
    w_copies = {}
    for ci in range(N_CHUNKS):
        g_copies[ci].wait()
        _normalize_chunk(bufs[ci % 2])
        w_copies[ci] = writeback(ci)
        if ci + 2 < N_CHUNKS:
            w_copies[ci].wait()
            g_copies[ci + 2] = gather(ci + 2)
    for ci in range(max(0, N_CHUNKS - 2), N_CHUNKS):
        w_copies[ci].wait()


@jax.jit
def _mp_embedding(idx_flat, weight):
    mesh = plsc.VectorSubcoreMesh(core_axis_name="c", subcore_axis_name="s")
    run = pl.kernel(
        _sc_body,
        out_type=jax.ShapeDtypeStruct((B_FLAT, EMB_DIM), jnp.float32),
        mesh=mesh,
        scratch_types=[
            pltpu.VMEM((B_PER_W,), jnp.int32),
            pltpu.VMEM((CHUNK, EMB_DIM), jnp.float32),
            pltpu.VMEM((CHUNK, EMB_DIM), jnp.float32),
            pltpu.SemaphoreType.DMA,
            pltpu.SemaphoreType.DMA,
            pltpu.SemaphoreType.DMA,
            pltpu.SemaphoreType.DMA,
        ],
    )
    return run(idx_flat, weight)


def kernel(x, weight):
    idx_flat = x.reshape(-1).astype(jnp.int32)
    out = _mp_embedding(idx_flat, weight)
    return out.reshape(BATCH, FIELDS, EMB_DIM)


# speedup vs baseline: 1.1853x; 1.1853x over previous
import jax
import jax.numpy as jnp
from jax import lax
from jax.experimental import pallas as pl
from jax.experimental.pallas import tpu as pltpu
from jax.experimental.pallas import tpu_sc as plsc

NUM_EMB = 1000000
EMB_DIM = 32
BATCH = 16384
FIELDS = 26

NC = 2
NS = 16
NW = NC * NS
LANES = 16

B_FLAT = BATCH * FIELDS
B_PER_W = B_FLAT // NW
CHUNK = 1024
N_CHUNKS = B_PER_W // CHUNK
GROUPS = CHUNK // LANES

EPS = 1e-4
INV_SQRT_DIM = float(1.0 / (EMB_DIM ** 0.5))


def _normalize_chunk(buf):
    def group_body(g, carry):
        rid = g * LANES + lax.iota(jnp.int32, LANES)
        cols = []
        sq = jnp.zeros((LANES,), jnp.float32)
        for c in range(EMB_DIM):
            cid = jnp.full((LANES,), c, jnp.int32)
            xc = plsc.load_gather(buf, [rid, cid])
            cols.append(xc)
            sq = sq + xc * xc
        sq_m = jnp.maximum(sq, 1e-30)
        i = plsc.bitcast(sq_m, jnp.int32)
        i = jnp.int32(0x5F3759DF) - lax.shift_right_logical(i, 1)
        y = plsc.bitcast(i, jnp.float32)
        h = 0.5 * sq_m
        for _ in range(3):
            y = y * (1.5 - h * y * y)
        sqrt_s = sq_m * y
        scale = 1.0 / (EPS + sqrt_s * INV_SQRT_DIM)
        for c in range(EMB_DIM):
            cid = jnp.full((LANES,), c, jnp.int32)
            plsc.store_scatter(buf, [rid, cid], cols[c] * scale)
        return carry

    lax.fori_loop(0, GROUPS, group_body, 0)


def _sc_body(idx_hbm, table_hbm, out_hbm, idx_v, buf0, buf1, gsem0, gsem1,
             wsem0, wsem1):
    wid = lax.axis_index("s") * NC + lax.axis_index("c")
    base = wid * B_PER_W
    pltpu.sync_copy(idx_hbm.at[pl.ds(base, B_PER_W)], idx_v)

    bufs = (buf0, buf1)
    gsems = (gsem0, gsem1)
    wsems = (wsem0, wsem1)

    def gather(ci):
        return pltpu.async_copy(
            table_hbm.at[idx_v.at[pl.ds(ci * CHUNK, CHUNK)]],
            bufs[ci % 2], gsems[ci % 2])

    def writeback(ci):
        return pltpu.async_copy(
            bufs[ci % 2], out_hbm.at[pl.ds(base + ci * CHUNK, CHUNK)],
            wsems[ci % 2])

    g_copies = {0: gather(0), 1: gather(1)}
    w_copies = {}
    for ci in range(N_CHUNKS):
        g_copies[ci].wait()
        _normalize_chunk(bufs[ci % 2])
        w_copies[ci] = writeback(ci)
        if ci + 2 < N_CHUNKS:
            w_copies[ci].wait()
            g_copies[ci + 2] = gather(ci + 2)
    for ci in range(max(0, N_CHUNKS - 2), N_CHUNKS):
        w_copies[ci].wait()


@jax.jit
def _mp_embedding(idx_flat, weight):
    mesh = plsc.VectorSubcoreMesh(core_axis_name="c", subcore_axis_name="s")
    run = pl.kernel(
        _sc_body,
        out_type=jax.ShapeDtypeStruct((B_FLAT, EMB_DIM), jnp.float32),
        mesh=mesh,
        scratch_types=[
            pltpu.VMEM((B_PER_W,), jnp.int32),
            pltpu.VMEM((CHUNK, EMB_DIM), jnp.float32),
            pltpu.VMEM((CHUNK, EMB_DIM), jnp.float32),
            pltpu.SemaphoreType.DMA,
            pltpu.SemaphoreType.DMA,
            pltpu.SemaphoreType.DMA,
            pltpu.SemaphoreType.DMA,
        ],
        compiler_params=pltpu.CompilerParams(
            needs_layout_passes=False, use_tc_tiling_on_sc=False),
    )
    return run(idx_flat, weight)


def kernel(x, weight):
    idx_flat = x.reshape(-1).astype(jnp.int32)
    out = _mp_embedding(idx_flat, weight)
    return out.reshape(BATCH, FIELDS, EMB_DIM)
